# Initial kernel scaffold; baseline (speedup 1.0000x reference)
import jax
import jax.numpy as jnp
from jax.experimental import pallas as pl


def kernel(x, cg_tilde, repids_in, repids_out):
    raise NotImplementedError("write your pallas kernel here")


# speedup vs baseline: 91.4168x; 91.4168x over previous
import functools

import jax
import jax.numpy as jnp
from jax import lax
from jax.experimental import pallas as pl
from jax.experimental.pallas import tpu as pltpu
from jax.experimental.pallas import tpu_sc as plsc

N_ROWS = 8192
REP = 1152
WCOLS = 128
NBLK = 9
K_TRIPLES = 10368
SC_WORKERS = 27
PER_W = K_TRIPLES // SC_WORKERS
CHUNK = 128
NCHUNK = PER_W // CHUNK
TB = 512


@functools.partial(
    pl.kernel,
    mesh=plsc.VectorSubcoreMesh(core_axis_name="c", subcore_axis_name="s"),
    out_type=jax.ShapeDtypeStruct((REP * WCOLS,), jnp.float32),
    scratch_types=[
        pltpu.VMEM((NCHUNK, CHUNK), jnp.float32),
        pltpu.VMEM((NCHUNK, CHUNK), jnp.int32),
        pltpu.VMEM((NCHUNK, CHUNK), jnp.int32),
        pltpu.VMEM((NCHUNK, CHUNK), jnp.int32),
        pltpu.SemaphoreType.DMA,
    ],
)
def _sc_build_w(cg_hbm, rin_hbm, rout_hbm, w_hbm, cg_v, rin_v, rout_v,
                idx_v, sem):
    nc = 2
    wid = lax.axis_index("s") * nc + lax.axis_index("c")

    @pl.when(wid < SC_WORKERS)
    def _():
        base = wid * PER_W
        for j in range(NCHUNK):
            off = base + j * CHUNK
            pltpu.sync_copy(cg_hbm.at[pl.ds(off, CHUNK)], cg_v.at[j])
            pltpu.sync_copy(rin_hbm.at[pl.ds(off, CHUNK)], rin_v.at[j])
            pltpu.sync_copy(rout_hbm.at[pl.ds(off, CHUNK)], rout_v.at[j])
        for j in range(NCHUNK):
            for i in range(CHUNK // 16):
                sl = pl.ds(i * 16, 16)
                idx_v[j, sl] = rin_v[j, sl] * WCOLS + rout_v[j, sl]
        for j in range(NCHUNK):
            pltpu.async_copy(cg_v.at[j], w_hbm.at[idx_v.at[j]], sem).wait()


def _tc_body(x_ref, w_ref, o_ref):
    lane = lax.broadcasted_iota(jnp.int32, (REP, WCOLS), 1)
    wm = jnp.where(lane < NBLK, w_ref[...], jnp.float32(0.0))
    y = jnp.dot(x_ref[...], wm, preferred_element_type=jnp.float32)
    o_ref[:, :WCOLS] = y
    o_ref[:, WCOLS:] = jnp.zeros((TB, REP - WCOLS), jnp.float32)


_tc_matmul = pl.pallas_call(
    _tc_body,
    grid=(N_ROWS // TB,),
    in_specs=[
        pl.BlockSpec((TB, REP), lambda i: (i, 0)),
        pl.BlockSpec((REP, WCOLS), lambda i: (0, 0)),
    ],
    out_specs=pl.BlockSpec((TB, REP), lambda i: (i, 0)),
    out_shape=jax.ShapeDtypeStruct((N_ROWS, REP), jnp.float32),
)


@jax.jit
def kernel(x, cg_tilde, repids_in, repids_out):
    w_flat = _sc_build_w(cg_tilde, repids_in, repids_out)
    w = w_flat.reshape(REP, WCOLS)
    return _tc_matmul(x, w)


# speedup vs baseline: 95.1406x; 1.0407x over previous
import functools

import jax
import jax.numpy as jnp
from jax import lax
from jax.experimental import pallas as pl
from jax.experimental.pallas import tpu as pltpu
from jax.experimental.pallas import tpu_sc as plsc

N_ROWS = 8192
REP = 1152
WCOLS = 128
NBLK = 9
K_TRIPLES = 10368
SC_WORKERS = 27
PER_W = K_TRIPLES // SC_WORKERS
CHUNK = 128
NCHUNK = PER_W // CHUNK
TB = 1024


@functools.partial(
    pl.kernel,
    mesh=plsc.VectorSubcoreMesh(core_axis_name="c", subcore_axis_name="s"),
    out_type=jax.ShapeDtypeStruct((REP * WCOLS,), jnp.float32),
    scratch_types=[
        pltpu.VMEM((NCHUNK, CHUNK), jnp.float32),
        pltpu.VMEM((NCHUNK, CHUNK), jnp.int32),
        pltpu.VMEM((NCHUNK, CHUNK), jnp.int32),
        pltpu.VMEM((NCHUNK, CHUNK), jnp.int32),
        pltpu.SemaphoreType.DMA,
    ],
)
def _sc_build_w(cg_hbm, rin_hbm, rout_hbm, w_hbm, cg_v, rin_v, rout_v,
                idx_v, sem):
    nc = 2
    wid = lax.axis_index("s") * nc + lax.axis_index("c")

    @pl.when(wid < SC_WORKERS)
    def _():
        base = wid * PER_W
        for j in range(NCHUNK):
            off = base + j * CHUNK
            pltpu.sync_copy(cg_hbm.at[pl.ds(off, CHUNK)], cg_v.at[j])
            pltpu.sync_copy(rin_hbm.at[pl.ds(off, CHUNK)], rin_v.at[j])
            pltpu.sync_copy(rout_hbm.at[pl.ds(off, CHUNK)], rout_v.at[j])
        for j in range(NCHUNK):
            for i in range(CHUNK // 16):
                sl = pl.ds(i * 16, 16)
                idx_v[j, sl] = rin_v[j, sl] * WCOLS + rout_v[j, sl]
        for j in range(NCHUNK):
            pltpu.async_copy(cg_v.at[j], w_hbm.at[idx_v.at[j]], sem).wait()


def _tc_body(x_ref, w_ref, o_ref):
    lane = lax.broadcasted_iota(jnp.int32, (REP, WCOLS), 1)
    wm = jnp.where(lane < NBLK, w_ref[...], jnp.float32(0.0))
    y = jnp.dot(x_ref[...], wm, preferred_element_type=jnp.float32)
    o_ref[:, :WCOLS] = y
    o_ref[:, WCOLS:] = jnp.zeros((TB, REP - WCOLS), jnp.float32)


_tc_matmul = pl.pallas_call(
    _tc_body,
    grid=(N_ROWS // TB,),
    in_specs=[
        pl.BlockSpec((TB, REP), lambda i: (i, 0)),
        pl.BlockSpec((REP, WCOLS), lambda i: (0, 0)),
    ],
    out_specs=pl.BlockSpec((TB, REP), lambda i: (i, 0)),
    out_shape=jax.ShapeDtypeStruct((N_ROWS, REP), jnp.float32),
)


@jax.jit
def kernel(x, cg_tilde, repids_in, repids_out):
    w_flat = _sc_build_w(cg_tilde, repids_in, repids_out)
    w = w_flat.reshape(REP, WCOLS)
    return _tc_matmul(x, w)


# speedup vs baseline: 102.7570x; 1.0801x over previous
import functools

import jax
import jax.numpy as jnp
from jax import lax
from jax.experimental import pallas as pl
from jax.experimental.pallas import tpu as pltpu
from jax.experimental.pallas import tpu_sc as plsc

N_ROWS = 8192
REP = 1152
WCOLS = 128
NBLK = 9
K_TRIPLES = 10368
SC_WORKERS = 27
PER_W = K_TRIPLES // SC_WORKERS
CHUNK = 128
NCHUNK = PER_W // CHUNK
TB = 2048


@functools.partial(
    pl.kernel,
    mesh=plsc.VectorSubcoreMesh(core_axis_name="c", subcore_axis_name="s"),
    out_type=jax.ShapeDtypeStruct((REP * WCOLS,), jnp.float32),
    scratch_types=[
        pltpu.VMEM((PER_W,), jnp.float32),
        pltpu.VMEM((PER_W,), jnp.int32),
        pltpu.VMEM((PER_W,), jnp.int32),
        pltpu.VMEM((NCHUNK, CHUNK), jnp.int32),
        pltpu.SemaphoreType.DMA,
        pltpu.SemaphoreType.DMA,
    ],
)
def _sc_build_w(cg_hbm, rin_hbm, rout_hbm, w_hbm, cg_v, rin_v, rout_v,
                idx_v, sem_in, sem_sc):
    nc = 2
    wid = lax.axis_index("s") * nc + lax.axis_index("c")

    @pl.when(wid < SC_WORKERS)
    def _():
        base = wid * PER_W
        cp_cg = pltpu.async_copy(cg_hbm.at[pl.ds(base, PER_W)], cg_v, sem_in)
        cp_ri = pltpu.async_copy(rin_hbm.at[pl.ds(base, PER_W)], rin_v, sem_in)
        cp_ro = pltpu.async_copy(rout_hbm.at[pl.ds(base, PER_W)], rout_v,
                                 sem_in)
        cp_cg.wait()
        cp_ri.wait()
        cp_ro.wait()
        for j in range(NCHUNK):
            for i in range(CHUNK // 16):
                src = pl.ds(j * CHUNK + i * 16, 16)
                idx_v[j, pl.ds(i * 16, 16)] = (rin_v[src] * WCOLS
                                               + rout_v[src])
        cps = [pltpu.async_copy(cg_v.at[pl.ds(j * CHUNK, CHUNK)],
                                w_hbm.at[idx_v.at[j]], sem_sc)
               for j in range(NCHUNK)]
        for cp in cps:
            cp.wait()


def _tc_body(x_ref, w_ref, o_ref):
    lane = lax.broadcasted_iota(jnp.int32, (REP, WCOLS), 1)
    wm = jnp.where(lane < NBLK, w_ref[...], jnp.float32(0.0))
    y = jnp.dot(x_ref[...], wm, preferred_element_type=jnp.float32)
    o_ref[:, :WCOLS] = y
    o_ref[:, WCOLS:] = jnp.zeros((TB, REP - WCOLS), jnp.float32)


_tc_matmul = pl.pallas_call(
    _tc_body,
    grid=(N_ROWS // TB,),
    in_specs=[
        pl.BlockSpec((TB, REP), lambda i: (i, 0)),
        pl.BlockSpec((REP, WCOLS), lambda i: (0, 0)),
    ],
    out_specs=pl.BlockSpec((TB, REP), lambda i: (i, 0)),
    out_shape=jax.ShapeDtypeStruct((N_ROWS, REP), jnp.float32),
)


@jax.jit
def kernel(x, cg_tilde, repids_in, repids_out):
    w_flat = _sc_build_w(cg_tilde, repids_in, repids_out)
    w = w_flat.reshape(REP, WCOLS)
    return _tc_matmul(x, w)


# speedup vs baseline: 125.4478x; 1.2208x over previous
import functools

import jax
import jax.numpy as jnp
from jax import lax
from jax.experimental import pallas as pl
from jax.experimental.pallas import tpu as pltpu
from jax.experimental.pallas import tpu_sc as plsc

N_ROWS = 8192
REP = 1152
WCOLS = 128
NBLK = 9
K_TRIPLES = 10368
NSUB = 16
PER_S = K_TRIPLES // NSUB
CHUNK = 128
NCHUNK = 6
PAD_S = NCHUNK * CHUNK
ROW_SPLIT = 512
SP_ROWS = 640
DUMP = SP_ROWS * WCOLS
SP_WORDS = DUMP + CHUNK
TB = 2048


@functools.partial(
    pl.kernel,
    mesh=plsc.VectorSubcoreMesh(core_axis_name="c", subcore_axis_name="s"),
    out_type=jax.ShapeDtypeStruct((REP * WCOLS,), jnp.float32),
    scratch_types=[
        pltpu.VMEM((PAD_S,), jnp.float32),
        pltpu.VMEM((PAD_S,), jnp.int32),
        pltpu.VMEM((PAD_S,), jnp.int32),
        pltpu.VMEM((NCHUNK, CHUNK), jnp.int32),
        pltpu.VMEM_SHARED((SP_WORDS,), jnp.float32),
        pltpu.SemaphoreType.DMA,
        pltpu.SemaphoreType.DMA,
    ],
)
def _sc_build_w(cg_hbm, rin_hbm, rout_hbm, w_hbm, cg_v, rin_v, rout_v,
                idx_v, w_sp, sem_in, sem_sc):
    c_idx = lax.axis_index("c")
    s_idx = lax.axis_index("s")
    base = s_idx * PER_S
    cp_cg = pltpu.async_copy(cg_hbm.at[pl.ds(base, PER_S)],
                             cg_v.at[pl.ds(0, PER_S)], sem_in)
    cp_ri = pltpu.async_copy(rin_hbm.at[pl.ds(base, PER_S)],
                             rin_v.at[pl.ds(0, PER_S)], sem_in)
    cp_ro = pltpu.async_copy(rout_hbm.at[pl.ds(base, PER_S)],
                             rout_v.at[pl.ds(0, PER_S)], sem_in)
    cp_cg.wait()
    cp_ri.wait()
    cp_ro.wait()
    lane16 = lax.iota(jnp.int32, 16)
    for cval, lo, hi in ((0, 0, ROW_SPLIT), (1, ROW_SPLIT, REP)):
        @pl.when(c_idx == cval)
        def _():
            for j in range(NCHUNK):
                for i in range(CHUNK // 16):
                    off = j * CHUNK + i * 16
                    sl = pl.ds(off, 16)
                    rin = rin_v[sl]
                    valid = ((off + lane16 < PER_S) & (rin >= lo)
                             & (rin < hi))
                    idx_v[j, pl.ds(i * 16, 16)] = jnp.where(
                        valid, (rin - lo) * WCOLS + rout_v[sl],
                        DUMP + i * 16 + lane16)
            cps = [pltpu.async_copy(cg_v.at[pl.ds(j * CHUNK, CHUNK)],
                                    w_sp.at[idx_v.at[j]], sem_sc)
                   for j in range(NCHUNK)]
            for cp in cps:
                cp.wait()
    plsc.subcore_barrier()
    for cval, lo, nrows in ((0, 0, ROW_SPLIT), (1, ROW_SPLIT, REP - ROW_SPLIT)):
        @pl.when(c_idx == cval)
        def _():
            span = (nrows // NSUB) * WCOLS
            pltpu.sync_copy(
                w_sp.at[pl.ds(s_idx * span, span)],
                w_hbm.at[pl.ds(lo * WCOLS + s_idx * span, span)])


def _tc_body(x_ref, w_ref, o_ref):
    lane = lax.broadcasted_iota(jnp.int32, (REP, WCOLS), 1)
    wm = jnp.where(lane < NBLK, w_ref[...], jnp.float32(0.0))
    y = jnp.dot(x_ref[...], wm, preferred_element_type=jnp.float32)
    o_ref[:, :WCOLS] = y
    o_ref[:, WCOLS:] = jnp.zeros((TB, REP - WCOLS), jnp.float32)


_tc_matmul = pl.pallas_call(
    _tc_body,
    grid=(N_ROWS // TB,),
    in_specs=[
        pl.BlockSpec((TB, REP), lambda i: (i, 0)),
        pl.BlockSpec((REP, WCOLS), lambda i: (0, 0)),
    ],
    out_specs=pl.BlockSpec((TB, REP), lambda i: (i, 0)),
    out_shape=jax.ShapeDtypeStruct((N_ROWS, REP), jnp.float32),
)


@jax.jit
def kernel(x, cg_tilde, repids_in, repids_out):
    w_flat = _sc_build_w(cg_tilde, repids_in, repids_out)
    w = w_flat.reshape(REP, WCOLS)
    return _tc_matmul(x, w)
